# Initial kernel scaffold; baseline (speedup 1.0000x reference)
import jax
import jax.numpy as jnp
from jax.experimental import pallas as pl


def kernel(points, feats, Wc1, g1, b1, Wc2, bc2, Wr1, gr, br, Wr2, br2):
    raise NotImplementedError("write your pallas kernel here")


# speedup vs baseline: 1.1182x; 1.1182x over previous
import functools

import jax
import jax.numpy as jnp
import numpy as np
from jax import lax
from jax.experimental import pallas as pl
from jax.experimental.pallas import tpu as pltpu
from jax.experimental.pallas import tpu_sc as plsc

B, N, C = 4, 65536, 128
NCLS, KPTS, MID = 3, 4096, 128
EPS = 1e-5
NB = 2048
FR, FC = 512, 128
SR, SC_ = 32, 128
NW = 32
SHARD = B * KPTS // NW


def _erfc_xla(z):
    one = jnp.float32(1.0)
    ax = jnp.abs(z)
    z2 = z * z
    p = jnp.float32(7.85386146e-05)
    for c in (-0.000801019371, 0.00518832775, -0.0268538129, 0.112835854,
              -0.37612626, 1.12837911):
        p = p * z2 + jnp.float32(c)
    res_a = one - z * p
    nz2 = -z2
    e = jnp.exp(nz2)
    q = e * (one / ax)
    w = one / z2
    r = w * jnp.float32(0.0232682) + jnp.float32(-0.138703942)
    for c in (0.368742466, -0.582473278, 0.621000469, -0.494451523, 0.340488,
              -0.274112701, 0.563825965):
        r = r * w + jnp.float32(c)
    s = w * jnp.float32(-10.477664) + jnp.float32(12.9772)
    for c in (-7.49551868, 2.92101908, -1.01526523, 0.42184633, -0.282076746,
              0.564189494):
        s = s * w + jnp.float32(c)
    poly = jnp.where(ax < jnp.float32(2.0), r, s)
    val = q * poly
    val = jnp.where(nz2 < jnp.float32(-88.7228394), jnp.float32(0.0), val)
    res_b = jnp.where(z < jnp.float32(0.0), jnp.float32(2.0) - val, val)
    return jnp.where(ax < one, res_a, res_b)


def _gelu_xla(x):
    z = (-x) * jnp.float32(0.707106769)
    return (x * jnp.float32(0.5)) * _erfc_xla(z)


def _heads_kernel(scale1_r, b1_r, wc1_r, wc2_r, bc2_r,
                  scaler_r, brr_r, wr1_r, wr2_r, br2_r,
                  f_r, cls_r, s_r, off_r):
    f = f_r[0]
    h = lax.dot_general(wc1_r[...], f, (((1,), (0,)), ((), ())),
                        preferred_element_type=jnp.float32)
    h = h * scale1_r[...] + b1_r[...]
    h = _gelu_xla(h)
    cls = lax.dot_general(wc2_r[...], h, (((1,), (0,)), ((), ())),
                          preferred_element_type=jnp.float32)
    cls = cls + bc2_r[...]
    cls_r[0] = cls
    s_r[0, 0] = jax.nn.sigmoid(jnp.max(cls, axis=0))
    hr = lax.dot_general(wr1_r[...], f, (((1,), (0,)), ((), ())),
                         preferred_element_type=jnp.float32)
    hr = hr * scaler_r[...] + brr_r[...]
    hr = _gelu_xla(hr)
    off = lax.dot_general(wr2_r[...], hr, (((1,), (0,)), ((), ())),
                          preferred_element_type=jnp.float32)
    off_r[0] = off + br2_r[...]


def _run_heads(feats, scale1, b1c, Wc1, Wc2, bc2c, scaler, brc, Wr1, Wr2, br2c):
    grid = (B, N // NB)
    zz = lambda b, n: (0, 0)
    return pl.pallas_call(
        _heads_kernel,
        grid=grid,
        in_specs=[
            pl.BlockSpec((C, 1), zz),
            pl.BlockSpec((C, 1), zz),
            pl.BlockSpec((C, C), zz),
            pl.BlockSpec((NCLS, C), zz),
            pl.BlockSpec((NCLS, 1), zz),
            pl.BlockSpec((MID, 1), zz),
            pl.BlockSpec((MID, 1), zz),
            pl.BlockSpec((MID, C), zz),
            pl.BlockSpec((NCLS, MID), zz),
            pl.BlockSpec((NCLS, 1), zz),
            pl.BlockSpec((1, C, NB), lambda b, n: (b, 0, n)),
        ],
        out_specs=[
            pl.BlockSpec((1, NCLS, NB), lambda b, n: (b, 0, n)),
            pl.BlockSpec((1, 1, NB), lambda b, n: (b, 0, n)),
            pl.BlockSpec((1, NCLS, NB), lambda b, n: (b, 0, n)),
        ],
        out_shape=[
            jax.ShapeDtypeStruct((B, NCLS, N), jnp.float32),
            jax.ShapeDtypeStruct((B, 1, N), jnp.float32),
            jax.ShapeDtypeStruct((B, NCLS, N), jnp.float32),
        ],
        compiler_params=pltpu.CompilerParams(
            dimension_semantics=("parallel", "arbitrary")),
    )(scale1, b1c, Wc1, Wc2, bc2c, scaler, brc, Wr1, Wr2, br2c, feats)


def _rot(x, s, axis):
    if s == 0:
        return x
    if axis == 1:
        return jnp.concatenate([x[:, s:], x[:, :s]], axis=1)
    return jnp.concatenate([x[s:], x[:s]], axis=0)


def _cumsum_flat(m):
    lane = lax.broadcasted_iota(jnp.int32, (FR, FC), 1)
    x = m
    s = 1
    while s < FC:
        sh = _rot(x, FC - s, 1)
        x = x + jnp.where(lane >= s, sh, 0)
        s *= 2
    tot = x[:, FC - 1:FC]
    sub = lax.broadcasted_iota(jnp.int32, (FR, 1), 0)
    t = tot
    s = 1
    while s < FR:
        sh = _rot(t, FR - s, 0)
        t = t + jnp.where(sub >= s, sh, 0)
        s *= 2
    return x + (t - tot)


def _flat_shift_left(x, b):
    if b % FC == 0:
        return _rot(x, b // FC, 0)
    bl = b % FC
    a = _rot(x, bl, 1)
    ab = _rot(a, b // FC + 1, 0)
    a = _rot(a, b // FC, 0) if b // FC else a
    lane = lax.broadcasted_iota(jnp.int32, (FR, FC), 1)
    return jnp.where(lane < FC - bl, a, ab)


def _topk_kernel(s_r, idx_r):
    k = lax.bitcast_convert_type(s_r[0], jnp.int32)
    kpts = jnp.int32(KPTS)

    def bs_body(_, carry):
        lo, hi = carry
        mid = lax.div(lo + hi, jnp.int32(2))
        cnt = jnp.sum((k > mid).astype(jnp.int32))
        take_hi = cnt >= kpts
        lo = jnp.where(take_hi, mid, lo)
        hi = jnp.where(take_hi, hi, mid)
        return lo, hi

    lo0 = jnp.int32(0)
    hi0 = jnp.int32(0x3F800000)
    _, kth = lax.fori_loop(0, 31, bs_body, (lo0, hi0))
    c_gt = jnp.sum((k > kth).astype(jnp.int32))
    r_ties = kpts - c_gt

    eq = k == kth
    cum2 = _cumsum_flat(eq.astype(jnp.int32))
    sel = (k > kth) | (eq & (cum2 <= r_ties))
    cum = _cumsum_flat(sel.astype(jnp.int32))

    fi = (lax.broadcasted_iota(jnp.int32, (FR, FC), 0) * FC
          + lax.broadcasted_iota(jnp.int32, (FR, FC), 1))
    sh = jnp.where(sel, fi - (cum - 1), 0)
    kv = jnp.where(sel, k, 0)
    ix = jnp.where(sel, fi, 0)
    b = 1
    while b < N:
        ksrc = _flat_shift_left(kv, b)
        isrc = _flat_shift_left(ix, b)
        ssrc = _flat_shift_left(sh, b)
        incoming = (jnp.bitwise_and(ssrc, b) != 0) & (fi < N - b)
        vacate = jnp.bitwise_and(sh, b) != 0
        zero = jnp.int32(0)
        kv = jnp.where(incoming, ksrc, jnp.where(vacate, zero, kv))
        ix = jnp.where(incoming, isrc, jnp.where(vacate, zero, ix))
        sh = jnp.where(incoming, ssrc, jnp.where(vacate, zero, sh))
        b *= 2

    kv32 = kv[:SR, :]
    ix32 = ix[:SR, :]
    flat_j = (lax.broadcasted_iota(jnp.int32, (SR, SC_), 0) * SC_
              + lax.broadcasted_iota(jnp.int32, (SR, SC_), 1))

    def partner(x, s):
        if s < SC_:
            left = _rot(x, s, 1)
            right = _rot(x, SC_ - s, 1)
        else:
            rs = s // SC_
            left = _rot(x, rs, 0)
            right = _rot(x, SR - rs, 0)
        return left, right

    kk = 2
    while kk <= KPTS:
        s = kk // 2
        while s >= 1:
            bit_s = jnp.bitwise_and(flat_j, s) == 0
            dir_asc = jnp.bitwise_and(flat_j, kk) == 0
            vl, vr = partner(kv32, s)
            il, ir = partner(ix32, s)
            bv = jnp.where(bit_s, vl, vr)
            bi = jnp.where(bit_s, il, ir)
            a_first = (kv32 > bv) | ((kv32 == bv) & (ix32 < bi))
            keep = a_first == (bit_s == dir_asc)
            kv32 = jnp.where(keep, kv32, bv)
            ix32 = jnp.where(keep, ix32, bi)
            s //= 2
        kk *= 2

    idx_r[0] = ix32


def _run_topk(s16):
    return pl.pallas_call(
        _topk_kernel,
        grid=(B,),
        in_specs=[pl.BlockSpec((1, FR, FC), lambda b: (b, 0, 0))],
        out_specs=pl.BlockSpec((1, SR, SC_), lambda b: (b, 0, 0)),
        out_shape=jax.ShapeDtypeStruct((B, SR, SC_), jnp.int32),
    )(s16)


def _sc_gather_kernel(px, py, pz, ox, oy, oz, idx_hbm,
                      gpx, gpy, gpz, gox, goy, goz,
                      idx_v, b0, b1_, b2, b3, b4, b5, sem):
    wid = lax.axis_index("s") * 2 + lax.axis_index("c")
    base = wid * SHARD
    pltpu.sync_copy(idx_hbm.at[pl.ds(base, SHARD)], idx_v)
    cps = []
    for tab, buf in ((px, b0), (py, b1_), (pz, b2),
                     (ox, b3), (oy, b4), (oz, b5)):
        cps.append(pltpu.async_copy(tab.at[idx_v], buf, sem))
    for cp in cps:
        cp.wait()
    for out, buf in ((gpx, b0), (gpy, b1_), (gpz, b2),
                     (gox, b3), (goy, b4), (goz, b5)):
        pltpu.sync_copy(buf, out.at[pl.ds(base, SHARD)])


def _run_sc_gather(planes, idx_flat):
    mesh = plsc.VectorSubcoreMesh(core_axis_name="c", subcore_axis_name="s")
    out_t = [jax.ShapeDtypeStruct((B * KPTS,), jnp.float32)] * 6
    scratch = ([pltpu.VMEM((SHARD,), jnp.int32)]
               + [pltpu.VMEM((SHARD,), jnp.float32)] * 6
               + [pltpu.SemaphoreType.DMA])
    fn = functools.partial(pl.kernel, mesh=mesh, out_type=out_t,
                           scratch_types=scratch)(_sc_gather_kernel)
    return fn(*planes, idx_flat)


def _combine_kernel(org_r, off_r, pred_r):
    plane = lax.broadcasted_iota(jnp.int32, (NCLS, B * KPTS), 0)
    lim = jnp.where(plane < 2, jnp.float32(3.0), jnp.float32(2.0))
    off = off_r[...]
    limited = jnp.where(off > lim, lim, off)
    limited = jnp.where(limited < -lim, -lim, limited)
    pred_r[...] = org_r[...] + limited


def _run_combine(org_p, off_p):
    return pl.pallas_call(
        _combine_kernel,
        out_shape=jax.ShapeDtypeStruct((NCLS, B * KPTS), jnp.float32),
    )(org_p, off_p)


def kernel(points, feats, Wc1, g1, b1, Wc2, bc2, Wr1, gr, br, Wr2, br2):
    scale1 = (g1 / jnp.sqrt(1.0 + EPS)).astype(jnp.float32).reshape(C, 1)
    b1c = b1.reshape(C, 1)
    bc2c = bc2.reshape(NCLS, 1)
    scaler = (gr / jnp.sqrt(1.0 + EPS)).astype(jnp.float32).reshape(MID, 1)
    brc = br.reshape(MID, 1)
    br2c = br2.reshape(NCLS, 1)

    cls_preds, scores, offs = _run_heads(
        feats, scale1, b1c, Wc1, Wc2, bc2c, scaler, brc, Wr1, Wr2, br2c)

    s512 = scores.reshape(B, FR, FC)
    idx32 = _run_topk(s512)
    sidx = idx32.reshape(B, KPTS)

    idx_flat = (sidx + (jnp.arange(B, dtype=jnp.int32) * N)[:, None]).reshape(-1)

    pts_p = jnp.transpose(points, (2, 0, 1)).reshape(NCLS, B * N)
    offs_p = jnp.transpose(offs, (1, 0, 2)).reshape(NCLS, B * N)
    planes = (pts_p[0], pts_p[1], pts_p[2], offs_p[0], offs_p[1], offs_p[2])
    g = _run_sc_gather(planes, idx_flat)
    org_p = jnp.stack(g[:3], axis=0)
    offsel_p = jnp.stack(g[3:], axis=0)

    pred_p = _run_combine(org_p, offsel_p)

    to_out = lambda a: a.reshape(NCLS, B, KPTS).transpose(1, 2, 0)
    return (cls_preds, to_out(pred_p), to_out(org_p), to_out(offsel_p))


# speedup vs baseline: 1.7583x; 1.5725x over previous
import functools

import jax
import jax.numpy as jnp
import numpy as np
from jax import lax
from jax.experimental import pallas as pl
from jax.experimental.pallas import tpu as pltpu
from jax.experimental.pallas import tpu_sc as plsc

B, N, C = 4, 65536, 128
NCLS, KPTS, MID = 3, 4096, 128
EPS = 1e-5
NB = 2048
FR, FC = 512, 128
SR, SC_ = 32, 128
NW = 32
SHARD = B * KPTS // NW
CPW = C // (NW // B)


def _erfc_xla(z):
    one = jnp.float32(1.0)
    ax = jnp.abs(z)
    z2 = z * z
    p = jnp.float32(7.85386146e-05)
    for c in (-0.000801019371, 0.00518832775, -0.0268538129, 0.112835854,
              -0.37612626, 1.12837911):
        p = p * z2 + jnp.float32(c)
    res_a = one - z * p
    nz2 = -z2
    e = jnp.exp(nz2)
    q = e * (one / ax)
    w = one / z2
    r = w * jnp.float32(0.0232682) + jnp.float32(-0.138703942)
    for c in (0.368742466, -0.582473278, 0.621000469, -0.494451523, 0.340488,
              -0.274112701, 0.563825965):
        r = r * w + jnp.float32(c)
    s = w * jnp.float32(-10.477664) + jnp.float32(12.9772)
    for c in (-7.49551868, 2.92101908, -1.01526523, 0.42184633, -0.282076746,
              0.564189494):
        s = s * w + jnp.float32(c)
    poly = jnp.where(ax < jnp.float32(2.0), r, s)
    val = q * poly
    val = jnp.where(nz2 < jnp.float32(-88.7228394), jnp.float32(0.0), val)
    res_b = jnp.where(z < jnp.float32(0.0), jnp.float32(2.0) - val, val)
    return jnp.where(ax < one, res_a, res_b)


def _gelu_xla(x):
    z = (-x) * jnp.float32(0.707106769)
    return (x * jnp.float32(0.5)) * _erfc_xla(z)


def _conf_kernel(scale1_r, b1_r, wc1_r, wc2_r, bc2_r, f_r, cls_r, s_r, ft_r):
    f = f_r[0]
    h = lax.dot_general(wc1_r[...], f, (((1,), (0,)), ((), ())),
                        preferred_element_type=jnp.float32)
    h = h * scale1_r[...] + b1_r[...]
    h = _gelu_xla(h)
    cls = lax.dot_general(wc2_r[...], h, (((1,), (0,)), ((), ())),
                          preferred_element_type=jnp.float32)
    cls = cls + bc2_r[...]
    cls_r[0] = cls
    s_r[0, 0] = jax.nn.sigmoid(jnp.max(cls, axis=0))
    ft_r[0] = jnp.transpose(f, (1, 0))


def _run_conf(feats, scale1, b1c, Wc1, Wc2, bc2c):
    zz = lambda b, n: (0, 0)
    return pl.pallas_call(
        _conf_kernel,
        grid=(B, N // NB),
        in_specs=[
            pl.BlockSpec((C, 1), zz),
            pl.BlockSpec((C, 1), zz),
            pl.BlockSpec((C, C), zz),
            pl.BlockSpec((NCLS, C), zz),
            pl.BlockSpec((NCLS, 1), zz),
            pl.BlockSpec((1, C, NB), lambda b, n: (b, 0, n)),
        ],
        out_specs=[
            pl.BlockSpec((1, NCLS, NB), lambda b, n: (b, 0, n)),
            pl.BlockSpec((1, 1, NB), lambda b, n: (b, 0, n)),
            pl.BlockSpec((1, NB, C), lambda b, n: (b, n, 0)),
        ],
        out_shape=[
            jax.ShapeDtypeStruct((B, NCLS, N), jnp.float32),
            jax.ShapeDtypeStruct((B, 1, N), jnp.float32),
            jax.ShapeDtypeStruct((B, N, C), jnp.float32),
        ],
        compiler_params=pltpu.CompilerParams(
            dimension_semantics=("parallel", "arbitrary")),
    )(scale1, b1c, Wc1, Wc2, bc2c, feats)


def _rot(x, s, axis):
    if s == 0:
        return x
    if axis == 1:
        return jnp.concatenate([x[:, s:], x[:, :s]], axis=1)
    return jnp.concatenate([x[s:], x[:s]], axis=0)


def _cumsum_flat(m):
    lane = lax.broadcasted_iota(jnp.int32, (FR, FC), 1)
    x = m
    s = 1
    while s < FC:
        sh = _rot(x, FC - s, 1)
        x = x + jnp.where(lane >= s, sh, 0)
        s *= 2
    tot = x[:, FC - 1:FC]
    sub = lax.broadcasted_iota(jnp.int32, (FR, 1), 0)
    t = tot
    s = 1
    while s < FR:
        sh = _rot(t, FR - s, 0)
        t = t + jnp.where(sub >= s, sh, 0)
        s *= 2
    return x + (t - tot)


def _flat_shift_left(x, b):
    if b % FC == 0:
        return _rot(x, b // FC, 0)
    bl = b % FC
    a = _rot(x, bl, 1)
    ab = _rot(a, b // FC + 1, 0)
    if b // FC:
        a = _rot(a, b // FC, 0)
    lane = lax.broadcasted_iota(jnp.int32, (FR, FC), 1)
    return jnp.where(lane < FC - bl, a, ab)


def _topk_kernel(s_r, idx_r):
    k = lax.bitcast_convert_type(s_r[0], jnp.int32)
    kpts = jnp.int32(KPTS)

    def bs_body(_, carry):
        lo, hi = carry
        mid = lax.div(lo + hi, jnp.int32(2))
        cnt = jnp.sum((k > mid).astype(jnp.int32))
        take_hi = cnt >= kpts
        lo = jnp.where(take_hi, mid, lo)
        hi = jnp.where(take_hi, hi, mid)
        return lo, hi

    _, kth = lax.fori_loop(0, 31, bs_body, (jnp.int32(0), jnp.int32(0x3F800000)))
    c_gt = jnp.sum((k > kth).astype(jnp.int32))
    r_ties = kpts - c_gt

    eq = k == kth
    cum2 = _cumsum_flat(eq.astype(jnp.int32))
    sel = (k > kth) | (eq & (cum2 <= r_ties))
    cum = _cumsum_flat(sel.astype(jnp.int32))

    fi = (lax.broadcasted_iota(jnp.int32, (FR, FC), 0) * FC
          + lax.broadcasted_iota(jnp.int32, (FR, FC), 1))
    sh = jnp.where(sel, fi - (cum - 1), 0)
    kv = jnp.where(sel, k, 0)
    ix = jnp.where(sel, fi, 0)
    b = 1
    while b < N:
        ksrc = _flat_shift_left(kv, b)
        isrc = _flat_shift_left(ix, b)
        ssrc = _flat_shift_left(sh, b)
        incoming = (jnp.bitwise_and(ssrc, b) != 0) & (fi < N - b)
        vacate = jnp.bitwise_and(sh, b) != 0
        zero = jnp.int32(0)
        kv = jnp.where(incoming, ksrc, jnp.where(vacate, zero, kv))
        ix = jnp.where(incoming, isrc, jnp.where(vacate, zero, ix))
        sh = jnp.where(incoming, ssrc, jnp.where(vacate, zero, sh))
        b *= 2

    kv32 = kv[:SR, :]
    ix32 = ix[:SR, :]
    flat_j = (lax.broadcasted_iota(jnp.int32, (SR, SC_), 0) * SC_
              + lax.broadcasted_iota(jnp.int32, (SR, SC_), 1))

    def partner(x, s):
        if s < SC_:
            left = _rot(x, s, 1)
            right = _rot(x, SC_ - s, 1)
        else:
            rs = s // SC_
            left = _rot(x, rs, 0)
            right = _rot(x, SR - rs, 0)
        return left, right

    kk = 2
    while kk <= KPTS:
        s = kk // 2
        while s >= 1:
            bit_s = jnp.bitwise_and(flat_j, s) == 0
            dir_asc = jnp.bitwise_and(flat_j, kk) == 0
            vl, vr = partner(kv32, s)
            il, ir = partner(ix32, s)
            bv = jnp.where(bit_s, vl, vr)
            bi = jnp.where(bit_s, il, ir)
            a_first = (kv32 > bv) | ((kv32 == bv) & (ix32 < bi))
            keep = a_first == (bit_s == dir_asc)
            kv32 = jnp.where(keep, kv32, bv)
            ix32 = jnp.where(keep, ix32, bi)
            s //= 2
        kk *= 2

    idx_r[0] = ix32


def _run_topk(s512):
    return pl.pallas_call(
        _topk_kernel,
        grid=(B,),
        in_specs=[pl.BlockSpec((1, FR, FC), lambda b: (b, 0, 0))],
        out_specs=pl.BlockSpec((1, SR, SC_), lambda b: (b, 0, 0)),
        out_shape=jax.ShapeDtypeStruct((B, SR, SC_), jnp.int32),
    )(s512)


def _sc_gather_kernel(px, py, pz, ft, idxo_hbm,
                      gpx, gpy, gpz, gf,
                      idxo_v, b0, b1_, b2, rows_v, sem, fsem):
    wid = lax.axis_index("s") * 2 + lax.axis_index("c")
    base = wid * SHARD
    pltpu.sync_copy(idxo_hbm.at[pl.ds(base, SHARD)], idxo_v)
    fcp = pltpu.async_copy(ft.at[idxo_v], rows_v, fsem)
    cps = []
    for tab, buf in ((px, b0), (py, b1_), (pz, b2)):
        cps.append(pltpu.async_copy(tab.at[idxo_v], buf, sem))
    for cp in cps:
        cp.wait()
    for out, buf in ((gpx, b0), (gpy, b1_), (gpz, b2)):
        pltpu.sync_copy(buf, out.at[pl.ds(base, SHARD)])
    fcp.wait()
    pltpu.sync_copy(rows_v, gf.at[pl.ds(base, SHARD)])


def _run_sc_gather(px, py, pz, feats_t, idx_off):
    mesh = plsc.VectorSubcoreMesh(core_axis_name="c", subcore_axis_name="s")
    out_t = [jax.ShapeDtypeStruct((B * KPTS,), jnp.float32)] * 3 + [
        jax.ShapeDtypeStruct((B * KPTS, C), jnp.float32)]
    scratch = ([pltpu.VMEM((SHARD,), jnp.int32)]
               + [pltpu.VMEM((SHARD,), jnp.float32)] * 3
               + [pltpu.VMEM((SHARD, C), jnp.float32)]
               + [pltpu.SemaphoreType.DMA, pltpu.SemaphoreType.DMA])
    fn = functools.partial(pl.kernel, mesh=mesh, out_type=out_t,
                           scratch_types=scratch)(_sc_gather_kernel)
    return fn(px, py, pz, feats_t, idx_off)


def _reg_kernel(scaler_r, brr_r, wr1_r, wr2_r, br2_r, g_r, org_r,
                off_r, pred_r):
    g = g_r[0]
    hr = lax.dot_general(g, wr1_r[...], (((1,), (1,)), ((), ())),
                         preferred_element_type=jnp.float32)
    hr = hr * scaler_r[...] + brr_r[...]
    hr = _gelu_xla(hr)
    off = lax.dot_general(hr, wr2_r[...], (((1,), (1,)), ((), ())),
                          preferred_element_type=jnp.float32)
    off = off + br2_r[...]
    off_r[0] = off
    lane = lax.broadcasted_iota(jnp.int32, (KPTS, NCLS), 1)
    lim = jnp.where(lane < 2, jnp.float32(3.0), jnp.float32(2.0))
    limited = jnp.where(off > lim, lim, off)
    limited = jnp.where(limited < -lim, -lim, limited)
    pred_r[0] = org_r[0] + limited


def _run_reg(gfeats, org, scaler, brc, Wr1, Wr2, br2c):
    zz = lambda b: (0, 0)
    return pl.pallas_call(
        _reg_kernel,
        grid=(B,),
        in_specs=[
            pl.BlockSpec((1, MID), zz),
            pl.BlockSpec((1, MID), zz),
            pl.BlockSpec((MID, C), zz),
            pl.BlockSpec((NCLS, MID), zz),
            pl.BlockSpec((1, NCLS), zz),
            pl.BlockSpec((1, KPTS, C), lambda b: (b, 0, 0)),
            pl.BlockSpec((1, KPTS, NCLS), lambda b: (b, 0, 0)),
        ],
        out_specs=[
            pl.BlockSpec((1, KPTS, NCLS), lambda b: (b, 0, 0)),
            pl.BlockSpec((1, KPTS, NCLS), lambda b: (b, 0, 0)),
        ],
        out_shape=[
            jax.ShapeDtypeStruct((B, KPTS, NCLS), jnp.float32),
            jax.ShapeDtypeStruct((B, KPTS, NCLS), jnp.float32),
        ],
    )(scaler, brc, Wr1, Wr2, br2c, gfeats, org)


def kernel(points, feats, Wc1, g1, b1, Wc2, bc2, Wr1, gr, br, Wr2, br2):
    scale1 = (g1 / jnp.sqrt(1.0 + EPS)).astype(jnp.float32).reshape(C, 1)
    b1c = b1.reshape(C, 1)
    bc2c = bc2.reshape(NCLS, 1)
    scaler = (gr / jnp.sqrt(1.0 + EPS)).astype(jnp.float32).reshape(1, MID)
    brc = br.reshape(1, MID)
    br2c = br2.reshape(1, NCLS)

    cls_preds, scores, feats_t = _run_conf(feats, scale1, b1c, Wc1, Wc2, bc2c)

    idx32 = _run_topk(scores.reshape(B, FR, FC))
    sidx = idx32.reshape(B, KPTS)

    idx_off = (sidx + (jnp.arange(B, dtype=jnp.int32) * N)[:, None]).reshape(-1)

    pts_p = jnp.transpose(points, (2, 0, 1)).reshape(NCLS, B * N)
    gpx, gpy, gpz, gfeats = _run_sc_gather(
        pts_p[0], pts_p[1], pts_p[2], feats_t.reshape(B * N, C), idx_off)

    org_p = jnp.stack([gpx, gpy, gpz], axis=0)
    ctr_origins = org_p.reshape(NCLS, B, KPTS).transpose(1, 2, 0)

    off_b, pred_b = _run_reg(gfeats.reshape(B, KPTS, C), ctr_origins,
                             scaler, brc, Wr1, Wr2, br2c)
    return (cls_preds, pred_b, ctr_origins, off_b)


# speedup vs baseline: 1.7699x; 1.0066x over previous
import functools

import jax
import jax.numpy as jnp
import numpy as np
from jax import lax
from jax.experimental import pallas as pl
from jax.experimental.pallas import tpu as pltpu
from jax.experimental.pallas import tpu_sc as plsc

B, N, C = 4, 65536, 128
NCLS, KPTS, MID = 3, 4096, 128
EPS = 1e-5
NB = 2048
FR, FC = 512, 128
SR, SC_ = 32, 128
NW = 32
SHARD = B * KPTS // NW
CPW = C // (NW // B)


def _erfc_xla(z):
    one = jnp.float32(1.0)
    ax = jnp.abs(z)
    z2 = z * z
    p = jnp.float32(7.85386146e-05)
    for c in (-0.000801019371, 0.00518832775, -0.0268538129, 0.112835854,
              -0.37612626, 1.12837911):
        p = p * z2 + jnp.float32(c)
    res_a = one - z * p
    nz2 = -z2
    e = jnp.exp(nz2)
    q = e * (one / ax)
    w = one / z2
    r = w * jnp.float32(0.0232682) + jnp.float32(-0.138703942)
    for c in (0.368742466, -0.582473278, 0.621000469, -0.494451523, 0.340488,
              -0.274112701, 0.563825965):
        r = r * w + jnp.float32(c)
    s = w * jnp.float32(-10.477664) + jnp.float32(12.9772)
    for c in (-7.49551868, 2.92101908, -1.01526523, 0.42184633, -0.282076746,
              0.564189494):
        s = s * w + jnp.float32(c)
    poly = jnp.where(ax < jnp.float32(2.0), r, s)
    val = q * poly
    val = jnp.where(nz2 < jnp.float32(-88.7228394), jnp.float32(0.0), val)
    res_b = jnp.where(z < jnp.float32(0.0), jnp.float32(2.0) - val, val)
    return jnp.where(ax < one, res_a, res_b)


def _gelu_xla(x):
    z = (-x) * jnp.float32(0.707106769)
    return (x * jnp.float32(0.5)) * _erfc_xla(z)


def _conf_kernel(scale1_r, b1_r, wc1_r, wc2_r, bc2_r, f_r, cls_r, s_r, ft_r):
    f = f_r[0]
    h = lax.dot_general(wc1_r[...], f, (((1,), (0,)), ((), ())),
                        preferred_element_type=jnp.float32)
    h = h * scale1_r[...] + b1_r[...]
    h = _gelu_xla(h)
    cls = lax.dot_general(wc2_r[...], h, (((1,), (0,)), ((), ())),
                          preferred_element_type=jnp.float32)
    cls = cls + bc2_r[...]
    cls_r[0] = cls
    s_r[0, 0] = jax.nn.sigmoid(jnp.max(cls, axis=0))
    ft_r[0] = jnp.transpose(f, (1, 0))


def _run_conf(feats, scale1, b1c, Wc1, Wc2, bc2c):
    zz = lambda b, n: (0, 0)
    return pl.pallas_call(
        _conf_kernel,
        grid=(B, N // NB),
        in_specs=[
            pl.BlockSpec((C, 1), zz),
            pl.BlockSpec((C, 1), zz),
            pl.BlockSpec((C, C), zz),
            pl.BlockSpec((NCLS, C), zz),
            pl.BlockSpec((NCLS, 1), zz),
            pl.BlockSpec((1, C, NB), lambda b, n: (b, 0, n)),
        ],
        out_specs=[
            pl.BlockSpec((1, NCLS, NB), lambda b, n: (b, 0, n)),
            pl.BlockSpec((1, 1, NB), lambda b, n: (b, 0, n)),
            pl.BlockSpec((1, NB, C), lambda b, n: (b, n, 0)),
        ],
        out_shape=[
            jax.ShapeDtypeStruct((B, NCLS, N), jnp.float32),
            jax.ShapeDtypeStruct((B, 1, N), jnp.float32),
            jax.ShapeDtypeStruct((B, N, C), jnp.float32),
        ],
        compiler_params=pltpu.CompilerParams(
            dimension_semantics=("parallel", "arbitrary")),
    )(scale1, b1c, Wc1, Wc2, bc2c, feats)


def _rot(x, s, axis):
    if s == 0:
        return x
    if axis == 1:
        return jnp.concatenate([x[:, s:], x[:, :s]], axis=1)
    return jnp.concatenate([x[s:], x[:s]], axis=0)


def _cumsum_flat(m):
    lane = lax.broadcasted_iota(jnp.int32, (FR, FC), 1)
    x = m
    s = 1
    while s < FC:
        sh = _rot(x, FC - s, 1)
        x = x + jnp.where(lane >= s, sh, 0)
        s *= 2
    tot = x[:, FC - 1:FC]
    sub = lax.broadcasted_iota(jnp.int32, (FR, 1), 0)
    t = tot
    s = 1
    while s < FR:
        sh = _rot(t, FR - s, 0)
        t = t + jnp.where(sub >= s, sh, 0)
        s *= 2
    return x + (t - tot)


def _flat_shift_left(x, b):
    if b % FC == 0:
        return _rot(x, b // FC, 0)
    bl = b % FC
    a = _rot(x, bl, 1)
    ab = _rot(a, b // FC + 1, 0)
    if b // FC:
        a = _rot(a, b // FC, 0)
    lane = lax.broadcasted_iota(jnp.int32, (FR, FC), 1)
    return jnp.where(lane < FC - bl, a, ab)


def _topk_kernel(s_r, idx_r):
    k = lax.bitcast_convert_type(s_r[0], jnp.int32)
    kpts = jnp.int32(KPTS)

    def bs_body(_, carry):
        lo, hi = carry
        mid = lax.div(lo + hi, jnp.int32(2))
        cnt = jnp.sum((k > mid).astype(jnp.int32))
        take_hi = cnt >= kpts
        lo = jnp.where(take_hi, mid, lo)
        hi = jnp.where(take_hi, hi, mid)
        return lo, hi

    _, kth = lax.fori_loop(0, 31, bs_body, (jnp.int32(0), jnp.int32(0x3F800000)))
    c_gt = jnp.sum((k > kth).astype(jnp.int32))
    r_ties = kpts - c_gt

    eq = k == kth
    cum2 = _cumsum_flat(eq.astype(jnp.int32))
    sel = (k > kth) | (eq & (cum2 <= r_ties))
    cum = _cumsum_flat(sel.astype(jnp.int32))

    fi = (lax.broadcasted_iota(jnp.int32, (FR, FC), 0) * FC
          + lax.broadcasted_iota(jnp.int32, (FR, FC), 1))
    sh = jnp.where(sel, fi - (cum - 1), 0)
    kv = jnp.where(sel, k, 0)
    b = 1
    while b < N:
        ksrc = _flat_shift_left(kv, b)
        ssrc = _flat_shift_left(sh, b)
        incoming = (jnp.bitwise_and(ssrc, b) != 0) & (fi < N - b)
        vacate = jnp.bitwise_and(sh, b) != 0
        zero = jnp.int32(0)
        kv = jnp.where(incoming, ksrc, jnp.where(vacate, zero, kv))
        sh = jnp.where(incoming, ssrc, jnp.where(vacate, zero, sh))
        b *= 2

    kv32 = kv[:SR, :]
    flat_j = (lax.broadcasted_iota(jnp.int32, (SR, SC_), 0) * SC_
              + lax.broadcasted_iota(jnp.int32, (SR, SC_), 1))
    ix32 = flat_j + sh[:SR, :]

    def partner(x, s):
        if s < SC_:
            left = _rot(x, s, 1)
            right = _rot(x, SC_ - s, 1)
        else:
            rs = s // SC_
            left = _rot(x, rs, 0)
            right = _rot(x, SR - rs, 0)
        return left, right

    kk = 2
    while kk <= KPTS:
        s = kk // 2
        while s >= 1:
            bit_s = jnp.bitwise_and(flat_j, s) == 0
            dir_asc = jnp.bitwise_and(flat_j, kk) == 0
            vl, vr = partner(kv32, s)
            il, ir = partner(ix32, s)
            bv = jnp.where(bit_s, vl, vr)
            bi = jnp.where(bit_s, il, ir)
            a_first = (kv32 > bv) | ((kv32 == bv) & (ix32 < bi))
            keep = a_first == (bit_s == dir_asc)
            kv32 = jnp.where(keep, kv32, bv)
            ix32 = jnp.where(keep, ix32, bi)
            s //= 2
        kk *= 2

    idx_r[0] = ix32


def _run_topk(s512):
    return pl.pallas_call(
        _topk_kernel,
        grid=(B,),
        in_specs=[pl.BlockSpec((1, FR, FC), lambda b: (b, 0, 0))],
        out_specs=pl.BlockSpec((1, SR, SC_), lambda b: (b, 0, 0)),
        out_shape=jax.ShapeDtypeStruct((B, SR, SC_), jnp.int32),
    )(s512)


def _sc_gather_kernel(px, py, pz, ft, idxo_hbm,
                      gpx, gpy, gpz, gf,
                      idxo_v, b0, b1_, b2, rows_v, sem, fsem):
    wid = lax.axis_index("s") * 2 + lax.axis_index("c")
    base = wid * SHARD
    pltpu.sync_copy(idxo_hbm.at[pl.ds(base, SHARD)], idxo_v)
    fcp = pltpu.async_copy(ft.at[idxo_v], rows_v, fsem)
    cps = []
    for tab, buf in ((px, b0), (py, b1_), (pz, b2)):
        cps.append(pltpu.async_copy(tab.at[idxo_v], buf, sem))
    for cp in cps:
        cp.wait()
    for out, buf in ((gpx, b0), (gpy, b1_), (gpz, b2)):
        pltpu.sync_copy(buf, out.at[pl.ds(base, SHARD)])
    fcp.wait()
    pltpu.sync_copy(rows_v, gf.at[pl.ds(base, SHARD)])


def _run_sc_gather(px, py, pz, feats_t, idx_off):
    mesh = plsc.VectorSubcoreMesh(core_axis_name="c", subcore_axis_name="s")
    out_t = [jax.ShapeDtypeStruct((B * KPTS,), jnp.float32)] * 3 + [
        jax.ShapeDtypeStruct((B * KPTS, C), jnp.float32)]
    scratch = ([pltpu.VMEM((SHARD,), jnp.int32)]
               + [pltpu.VMEM((SHARD,), jnp.float32)] * 3
               + [pltpu.VMEM((SHARD, C), jnp.float32)]
               + [pltpu.SemaphoreType.DMA, pltpu.SemaphoreType.DMA])
    fn = functools.partial(pl.kernel, mesh=mesh, out_type=out_t,
                           scratch_types=scratch)(_sc_gather_kernel)
    return fn(px, py, pz, feats_t, idx_off)


def _reg_kernel(scaler_r, brr_r, wr1_r, wr2_r, br2_r, g_r, org_r,
                off_r, pred_r):
    g = g_r[0]
    hr = lax.dot_general(g, wr1_r[...], (((1,), (1,)), ((), ())),
                         preferred_element_type=jnp.float32)
    hr = hr * scaler_r[...] + brr_r[...]
    hr = _gelu_xla(hr)
    off = lax.dot_general(hr, wr2_r[...], (((1,), (1,)), ((), ())),
                          preferred_element_type=jnp.float32)
    off = off + br2_r[...]
    off_r[0] = off
    lane = lax.broadcasted_iota(jnp.int32, (KPTS, NCLS), 1)
    lim = jnp.where(lane < 2, jnp.float32(3.0), jnp.float32(2.0))
    limited = jnp.where(off > lim, lim, off)
    limited = jnp.where(limited < -lim, -lim, limited)
    pred_r[0] = org_r[0] + limited


def _run_reg(gfeats, org, scaler, brc, Wr1, Wr2, br2c):
    zz = lambda b: (0, 0)
    return pl.pallas_call(
        _reg_kernel,
        grid=(B,),
        in_specs=[
            pl.BlockSpec((1, MID), zz),
            pl.BlockSpec((1, MID), zz),
            pl.BlockSpec((MID, C), zz),
            pl.BlockSpec((NCLS, MID), zz),
            pl.BlockSpec((1, NCLS), zz),
            pl.BlockSpec((1, KPTS, C), lambda b: (b, 0, 0)),
            pl.BlockSpec((1, KPTS, NCLS), lambda b: (b, 0, 0)),
        ],
        out_specs=[
            pl.BlockSpec((1, KPTS, NCLS), lambda b: (b, 0, 0)),
            pl.BlockSpec((1, KPTS, NCLS), lambda b: (b, 0, 0)),
        ],
        out_shape=[
            jax.ShapeDtypeStruct((B, KPTS, NCLS), jnp.float32),
            jax.ShapeDtypeStruct((B, KPTS, NCLS), jnp.float32),
        ],
    )(scaler, brc, Wr1, Wr2, br2c, gfeats, org)


def kernel(points, feats, Wc1, g1, b1, Wc2, bc2, Wr1, gr, br, Wr2, br2):
    scale1 = (g1 / jnp.sqrt(1.0 + EPS)).astype(jnp.float32).reshape(C, 1)
    b1c = b1.reshape(C, 1)
    bc2c = bc2.reshape(NCLS, 1)
    scaler = (gr / jnp.sqrt(1.0 + EPS)).astype(jnp.float32).reshape(1, MID)
    brc = br.reshape(1, MID)
    br2c = br2.reshape(1, NCLS)

    cls_preds, scores, feats_t = _run_conf(feats, scale1, b1c, Wc1, Wc2, bc2c)

    idx32 = _run_topk(scores.reshape(B, FR, FC))
    sidx = idx32.reshape(B, KPTS)

    idx_off = (sidx + (jnp.arange(B, dtype=jnp.int32) * N)[:, None]).reshape(-1)

    pts_p = jnp.transpose(points, (2, 0, 1)).reshape(NCLS, B * N)
    gpx, gpy, gpz, gfeats = _run_sc_gather(
        pts_p[0], pts_p[1], pts_p[2], feats_t.reshape(B * N, C), idx_off)

    org_p = jnp.stack([gpx, gpy, gpz], axis=0)
    ctr_origins = org_p.reshape(NCLS, B, KPTS).transpose(1, 2, 0)

    off_b, pred_b = _run_reg(gfeats.reshape(B, KPTS, C), ctr_origins,
                             scaler, brc, Wr1, Wr2, br2c)
    return (cls_preds, pred_b, ctr_origins, off_b)


# speedup vs baseline: 1.7713x; 1.0008x over previous
import functools

import jax
import jax.numpy as jnp
import numpy as np
from jax import lax
from jax.experimental import pallas as pl
from jax.experimental.pallas import tpu as pltpu
from jax.experimental.pallas import tpu_sc as plsc

B, N, C = 4, 65536, 128
NCLS, KPTS, MID = 3, 4096, 128
EPS = 1e-5
NB = 2048
FR, FC = 512, 128
SR, SC_ = 32, 128
NW = 32
SHARD = B * KPTS // NW
CPW = C // (NW // B)


def _erfc_xla(z):
    one = jnp.float32(1.0)
    ax = jnp.abs(z)
    z2 = z * z
    p = jnp.float32(7.85386146e-05)
    for c in (-0.000801019371, 0.00518832775, -0.0268538129, 0.112835854,
              -0.37612626, 1.12837911):
        p = p * z2 + jnp.float32(c)
    res_a = one - z * p
    nz2 = -z2
    e = jnp.exp(nz2)
    q = e * (one / ax)
    w = one / z2
    r = w * jnp.float32(0.0232682) + jnp.float32(-0.138703942)
    for c in (0.368742466, -0.582473278, 0.621000469, -0.494451523, 0.340488,
              -0.274112701, 0.563825965):
        r = r * w + jnp.float32(c)
    s = w * jnp.float32(-10.477664) + jnp.float32(12.9772)
    for c in (-7.49551868, 2.92101908, -1.01526523, 0.42184633, -0.282076746,
              0.564189494):
        s = s * w + jnp.float32(c)
    poly = jnp.where(ax < jnp.float32(2.0), r, s)
    val = q * poly
    val = jnp.where(nz2 < jnp.float32(-88.7228394), jnp.float32(0.0), val)
    res_b = jnp.where(z < jnp.float32(0.0), jnp.float32(2.0) - val, val)
    return jnp.where(ax < one, res_a, res_b)


def _gelu_xla(x):
    z = (-x) * jnp.float32(0.707106769)
    return (x * jnp.float32(0.5)) * _erfc_xla(z)


def _conf_kernel(scale1_r, b1_r, wc1_r, wc2_r, bc2_r, f_r, cls_r, s_r, ft_r):
    f = f_r[0]
    h = lax.dot_general(wc1_r[...], f, (((1,), (0,)), ((), ())),
                        preferred_element_type=jnp.float32)
    h = h * scale1_r[...] + b1_r[...]
    h = _gelu_xla(h)
    cls = lax.dot_general(wc2_r[...], h, (((1,), (0,)), ((), ())),
                          preferred_element_type=jnp.float32)
    cls = cls + bc2_r[...]
    cls_r[0] = cls
    s_r[0] = jax.nn.sigmoid(jnp.max(cls, axis=0)).reshape(NB // FC, FC)
    ft_r[0] = jnp.transpose(f, (1, 0))


def _run_conf(feats, scale1, b1c, Wc1, Wc2, bc2c):
    zz = lambda b, n: (0, 0)
    return pl.pallas_call(
        _conf_kernel,
        grid=(B, N // NB),
        in_specs=[
            pl.BlockSpec((C, 1), zz),
            pl.BlockSpec((C, 1), zz),
            pl.BlockSpec((C, C), zz),
            pl.BlockSpec((NCLS, C), zz),
            pl.BlockSpec((NCLS, 1), zz),
            pl.BlockSpec((1, C, NB), lambda b, n: (b, 0, n)),
        ],
        out_specs=[
            pl.BlockSpec((1, NCLS, NB), lambda b, n: (b, 0, n)),
            pl.BlockSpec((1, NB // FC, FC), lambda b, n: (b, n, 0)),
            pl.BlockSpec((1, NB, C), lambda b, n: (b, n, 0)),
        ],
        out_shape=[
            jax.ShapeDtypeStruct((B, NCLS, N), jnp.float32),
            jax.ShapeDtypeStruct((B, FR, FC), jnp.float32),
            jax.ShapeDtypeStruct((B, N, C), jnp.float32),
        ],
        compiler_params=pltpu.CompilerParams(
            dimension_semantics=("parallel", "arbitrary")),
    )(scale1, b1c, Wc1, Wc2, bc2c, feats)


def _rot(x, s, axis):
    if s == 0:
        return x
    if axis == 1:
        return jnp.concatenate([x[:, s:], x[:, :s]], axis=1)
    return jnp.concatenate([x[s:], x[:s]], axis=0)


def _cumsum_flat(m):
    lane = lax.broadcasted_iota(jnp.int32, (FR, FC), 1)
    x = m
    s = 1
    while s < FC:
        sh = _rot(x, FC - s, 1)
        x = x + jnp.where(lane >= s, sh, 0)
        s *= 2
    tot = x[:, FC - 1:FC]
    sub = lax.broadcasted_iota(jnp.int32, (FR, 1), 0)
    t = tot
    s = 1
    while s < FR:
        sh = _rot(t, FR - s, 0)
        t = t + jnp.where(sub >= s, sh, 0)
        s *= 2
    return x + (t - tot)


def _flat_shift_left(x, b):
    if b % FC == 0:
        return _rot(x, b // FC, 0)
    bl = b % FC
    a = _rot(x, bl, 1)
    ab = _rot(a, b // FC + 1, 0)
    if b // FC:
        a = _rot(a, b // FC, 0)
    lane = lax.broadcasted_iota(jnp.int32, (FR, FC), 1)
    return jnp.where(lane < FC - bl, a, ab)


def _topk_kernel(s_r, idx_r):
    k = lax.bitcast_convert_type(s_r[0], jnp.int32)
    kpts = jnp.int32(KPTS)

    def bs_body(_, carry):
        lo, hi = carry
        mid = lax.div(lo + hi, jnp.int32(2))
        cnt = jnp.sum((k > mid).astype(jnp.int32))
        take_hi = cnt >= kpts
        lo = jnp.where(take_hi, mid, lo)
        hi = jnp.where(take_hi, hi, mid)
        return lo, hi

    _, kth = lax.fori_loop(0, 31, bs_body, (jnp.int32(0), jnp.int32(0x3F800000)))
    c_gt = jnp.sum((k > kth).astype(jnp.int32))
    r_ties = kpts - c_gt

    eq = k == kth
    cum2 = _cumsum_flat(eq.astype(jnp.int32))
    sel = (k > kth) | (eq & (cum2 <= r_ties))
    cum = _cumsum_flat(sel.astype(jnp.int32))

    fi = (lax.broadcasted_iota(jnp.int32, (FR, FC), 0) * FC
          + lax.broadcasted_iota(jnp.int32, (FR, FC), 1))
    sh = jnp.where(sel, fi - (cum - 1), 0)
    kv = jnp.where(sel, k, 0)
    b = 1
    while b < N:
        ksrc = _flat_shift_left(kv, b)
        ssrc = _flat_shift_left(sh, b)
        incoming = (jnp.bitwise_and(ssrc, b) != 0) & (fi < N - b)
        vacate = jnp.bitwise_and(sh, b) != 0
        zero = jnp.int32(0)
        kv = jnp.where(incoming, ksrc, jnp.where(vacate, zero, kv))
        sh = jnp.where(incoming, ssrc, jnp.where(vacate, zero, sh))
        b *= 2

    kv32 = kv[:SR, :]
    flat_j = (lax.broadcasted_iota(jnp.int32, (SR, SC_), 0) * SC_
              + lax.broadcasted_iota(jnp.int32, (SR, SC_), 1))
    ix32 = flat_j + sh[:SR, :]

    def partner(x, s):
        if s < SC_:
            left = _rot(x, s, 1)
            right = _rot(x, SC_ - s, 1)
        else:
            rs = s // SC_
            left = _rot(x, rs, 0)
            right = _rot(x, SR - rs, 0)
        return left, right

    kk = 2
    while kk <= KPTS:
        s = kk // 2
        while s >= 1:
            bit_s = jnp.bitwise_and(flat_j, s) == 0
            dir_asc = jnp.bitwise_and(flat_j, kk) == 0
            vl, vr = partner(kv32, s)
            il, ir = partner(ix32, s)
            bv = jnp.where(bit_s, vl, vr)
            bi = jnp.where(bit_s, il, ir)
            a_first = (kv32 > bv) | ((kv32 == bv) & (ix32 < bi))
            keep = a_first == (bit_s == dir_asc)
            kv32 = jnp.where(keep, kv32, bv)
            ix32 = jnp.where(keep, ix32, bi)
            s //= 2
        kk *= 2

    idx_r[0] = ix32


def _run_topk(s512):
    return pl.pallas_call(
        _topk_kernel,
        grid=(B,),
        in_specs=[pl.BlockSpec((1, FR, FC), lambda b: (b, 0, 0))],
        out_specs=pl.BlockSpec((1, SR, SC_), lambda b: (b, 0, 0)),
        out_shape=jax.ShapeDtypeStruct((B, SR, SC_), jnp.int32),
    )(s512)


def _sc_gather_kernel(px, py, pz, ft, idxo_hbm,
                      gpx, gpy, gpz, gf,
                      idxo_v, b0, b1_, b2, rows_v, sem, fsem):
    wid = lax.axis_index("s") * 2 + lax.axis_index("c")
    base = wid * SHARD
    pltpu.sync_copy(idxo_hbm.at[pl.ds(base, SHARD)], idxo_v)
    fcp = pltpu.async_copy(ft.at[idxo_v], rows_v, fsem)
    cps = []
    for tab, buf in ((px, b0), (py, b1_), (pz, b2)):
        cps.append(pltpu.async_copy(tab.at[idxo_v], buf, sem))
    for cp in cps:
        cp.wait()
    for out, buf in ((gpx, b0), (gpy, b1_), (gpz, b2)):
        pltpu.sync_copy(buf, out.at[pl.ds(base, SHARD)])
    fcp.wait()
    pltpu.sync_copy(rows_v, gf.at[pl.ds(base, SHARD)])


def _run_sc_gather(px, py, pz, feats_t, idx_off):
    mesh = plsc.VectorSubcoreMesh(core_axis_name="c", subcore_axis_name="s")
    out_t = [jax.ShapeDtypeStruct((B * KPTS,), jnp.float32)] * 3 + [
        jax.ShapeDtypeStruct((B * KPTS, C), jnp.float32)]
    scratch = ([pltpu.VMEM((SHARD,), jnp.int32)]
               + [pltpu.VMEM((SHARD,), jnp.float32)] * 3
               + [pltpu.VMEM((SHARD, C), jnp.float32)]
               + [pltpu.SemaphoreType.DMA, pltpu.SemaphoreType.DMA])
    fn = functools.partial(pl.kernel, mesh=mesh, out_type=out_t,
                           scratch_types=scratch)(_sc_gather_kernel)
    return fn(px, py, pz, feats_t, idx_off)


def _reg_kernel(scaler_r, brr_r, wr1_r, wr2_r, br2_r, g_r, org_r,
                off_r, pred_r):
    g = g_r[0]
    hr = lax.dot_general(g, wr1_r[...], (((1,), (1,)), ((), ())),
                         preferred_element_type=jnp.float32)
    hr = hr * scaler_r[...] + brr_r[...]
    hr = _gelu_xla(hr)
    off = lax.dot_general(hr, wr2_r[...], (((1,), (1,)), ((), ())),
                          preferred_element_type=jnp.float32)
    off = off + br2_r[...]
    off_r[0] = off
    lane = lax.broadcasted_iota(jnp.int32, (KPTS, NCLS), 1)
    lim = jnp.where(lane < 2, jnp.float32(3.0), jnp.float32(2.0))
    limited = jnp.where(off > lim, lim, off)
    limited = jnp.where(limited < -lim, -lim, limited)
    pred_r[0] = org_r[0] + limited


def _run_reg(gfeats, org, scaler, brc, Wr1, Wr2, br2c):
    zz = lambda b: (0, 0)
    return pl.pallas_call(
        _reg_kernel,
        grid=(B,),
        in_specs=[
            pl.BlockSpec((1, MID), zz),
            pl.BlockSpec((1, MID), zz),
            pl.BlockSpec((MID, C), zz),
            pl.BlockSpec((NCLS, MID), zz),
            pl.BlockSpec((1, NCLS), zz),
            pl.BlockSpec((1, KPTS, C), lambda b: (b, 0, 0)),
            pl.BlockSpec((1, KPTS, NCLS), lambda b: (b, 0, 0)),
        ],
        out_specs=[
            pl.BlockSpec((1, KPTS, NCLS), lambda b: (b, 0, 0)),
            pl.BlockSpec((1, KPTS, NCLS), lambda b: (b, 0, 0)),
        ],
        out_shape=[
            jax.ShapeDtypeStruct((B, KPTS, NCLS), jnp.float32),
            jax.ShapeDtypeStruct((B, KPTS, NCLS), jnp.float32),
        ],
    )(scaler, brc, Wr1, Wr2, br2c, gfeats, org)


def kernel(points, feats, Wc1, g1, b1, Wc2, bc2, Wr1, gr, br, Wr2, br2):
    scale1 = (g1 / jnp.sqrt(1.0 + EPS)).astype(jnp.float32).reshape(C, 1)
    b1c = b1.reshape(C, 1)
    bc2c = bc2.reshape(NCLS, 1)
    scaler = (gr / jnp.sqrt(1.0 + EPS)).astype(jnp.float32).reshape(1, MID)
    brc = br.reshape(1, MID)
    br2c = br2.reshape(1, NCLS)

    cls_preds, scores, feats_t = _run_conf(feats, scale1, b1c, Wc1, Wc2, bc2c)

    idx32 = _run_topk(scores)
    sidx = idx32.reshape(B, KPTS)

    idx_off = (sidx + (jnp.arange(B, dtype=jnp.int32) * N)[:, None]).reshape(-1)

    pts_p = jnp.transpose(points, (2, 0, 1)).reshape(NCLS, B * N)
    gpx, gpy, gpz, gfeats = _run_sc_gather(
        pts_p[0], pts_p[1], pts_p[2], feats_t.reshape(B * N, C), idx_off)

    org_p = jnp.stack([gpx, gpy, gpz], axis=0)
    ctr_origins = org_p.reshape(NCLS, B, KPTS).transpose(1, 2, 0)

    off_b, pred_b = _run_reg(gfeats.reshape(B, KPTS, C), ctr_origins,
                             scaler, brc, Wr1, Wr2, br2c)
    return (cls_preds, pred_b, ctr_origins, off_b)
